# Initial kernel scaffold; baseline (speedup 1.0000x reference)
import jax
import jax.numpy as jnp
from jax.experimental import pallas as pl


def kernel(x, edge_index, lin_in_W, lin_in_b, W1, bn_gamma, bn_beta, lin_out_W, lin_out_b):
    raise NotImplementedError("write your pallas kernel here")


# speedup vs baseline: 5.4102x; 5.4102x over previous
import functools

import jax
import jax.numpy as jnp
import numpy as np
from jax import lax
from jax.experimental import pallas as pl
from jax.experimental.pallas import tpu as pltpu
from jax.experimental.pallas import tpu_sc as plsc

N = 10000
E = 160000
GENE = 256
H = 256
HH = 128
L = 8
NCLS = 16
ALPHA = 0.1
THETA = 0.5
EPS = 1e-5

NCORE = 2
NSUB = 16
LANE = 16

CHUNK = 128
E_PAD = 163840
EPC = E_PAD // NSUB
NCHUNK = EPC // CHUNK
N_ACC = 10240
RSUB = N_ACC // NSUB
TRASH = N_ACC - 1

DEG_NCHUNK = (E_PAD // 2) // NSUB // CHUNK

_mesh = plsc.VectorSubcoreMesh(
    core_axis_name="c", subcore_axis_name="s", num_cores=NCORE, num_subcores=NSUB)


def _sc_deg_body(dst2_hbm, out_hbm, dst_v, ones_v, zbuf, acc_sh):
    c = lax.axis_index("c")
    s = lax.axis_index("s")
    one16 = jnp.ones((LANE,), jnp.float32)
    zero16 = jnp.zeros((LANE,), jnp.float32)

    def initrow(i, _):
        ones_v[i, :] = one16
        return 0
    lax.fori_loop(0, CHUNK, initrow, 0)

    def zrow(i, _):
        zbuf[i, :] = zero16
        return 0
    lax.fori_loop(0, 64, zrow, 0)

    def zcp(k, _):
        pltpu.sync_copy(zbuf, acc_sh.at[pl.ds(s * RSUB + k * 64, 64)])
        return 0
    lax.fori_loop(0, RSUB // 64, zcp, 0)
    plsc.subcore_barrier()

    pltpu.sync_copy(
        dst2_hbm.at[pl.ds((c * NSUB + s) * DEG_NCHUNK, DEG_NCHUNK)], dst_v)

    def body(j, _):
        pltpu.sync_copy(ones_v, acc_sh.at[dst_v.at[j]], add=True)
        return 0
    lax.fori_loop(0, DEG_NCHUNK, body, 0)

    plsc.subcore_barrier()
    pltpu.sync_copy(acc_sh.at[pl.ds(s * RSUB, RSUB)],
                    out_hbm.at[c, pl.ds(s * RSUB, RSUB)])


_sc_deg = functools.partial(
    pl.kernel,
    out_type=jax.ShapeDtypeStruct((NCORE, N_ACC, LANE), jnp.float32),
    mesh=_mesh,
    scratch_types=[
        pltpu.VMEM((DEG_NCHUNK, CHUNK), jnp.int32),
        pltpu.VMEM((CHUNK, LANE), jnp.float32),
        pltpu.VMEM((64, LANE), jnp.float32),
        pltpu.VMEM_SHARED((N_ACC, LANE), jnp.float32),
    ],
)(_sc_deg_body)


def _sc_agg_body(g_hbm, src2_hbm, dst2_hbm, out_hbm,
                 src_v, dst_v, rows0, rows1, sem0, sem1, acc_sh):
    c = lax.axis_index("c")
    s = lax.axis_index("s")
    zero16 = jnp.zeros((LANE,), jnp.float32)

    def zrow(i, _):
        def zlane(j, _):
            rows0[i, pl.ds(j * LANE, LANE)] = zero16
            return 0
        return lax.fori_loop(0, HH // LANE, zlane, 0)
    lax.fori_loop(0, CHUNK, zrow, 0)

    def zcp(k, _):
        pltpu.sync_copy(rows0, acc_sh.at[pl.ds(s * RSUB + k * CHUNK, CHUNK)])
        return 0
    lax.fori_loop(0, RSUB // CHUNK, zcp, 0)
    plsc.subcore_barrier()

    half = NCHUNK // 2

    def phase(h, _):
        pltpu.sync_copy(
            src2_hbm.at[pl.ds((c * NSUB + s) * NCHUNK + h * half, half)], src_v)
        pltpu.sync_copy(
            dst2_hbm.at[pl.ds(s * NCHUNK + h * half, half)], dst_v)

        def body(i, _):
            j0 = 2 * i
            j1 = 2 * i + 1
            d0 = pltpu.async_copy(g_hbm.at[src_v.at[j0]], rows0, sem0)
            d1 = pltpu.async_copy(g_hbm.at[src_v.at[j1]], rows1, sem1)
            d0.wait()
            pltpu.sync_copy(rows0, acc_sh.at[dst_v.at[j0]], add=True)
            d1.wait()
            pltpu.sync_copy(rows1, acc_sh.at[dst_v.at[j1]], add=True)
            return 0
        return lax.fori_loop(0, half // 2, body, 0)
    lax.fori_loop(0, 2, phase, 0)

    plsc.subcore_barrier()
    pltpu.sync_copy(acc_sh.at[pl.ds(s * RSUB, RSUB)],
                    out_hbm.at[c, pl.ds(s * RSUB, RSUB)])


_sc_agg = functools.partial(
    pl.kernel,
    out_type=jax.ShapeDtypeStruct((NCORE, N_ACC, HH), jnp.float32),
    mesh=_mesh,
    scratch_types=[
        pltpu.VMEM((NCHUNK // 2, CHUNK), jnp.int32),
        pltpu.VMEM((NCHUNK // 2, CHUNK), jnp.int32),
        pltpu.VMEM((CHUNK, HH), jnp.float32),
        pltpu.VMEM((CHUNK, HH), jnp.float32),
        pltpu.SemaphoreType.DMA,
        pltpu.SemaphoreType.DMA,
        pltpu.VMEM_SHARED((N_ACC, HH), jnp.float32),
    ],
)(_sc_agg_body)


BLK = 1000
GRID = N // BLK

_INV_SQRT = float(1.0 / np.sqrt(1.0 + EPS))


def _tc_prologue_body(x_ref, w_ref, b_ref, degs_ref, x0_ref, dinv_ref, g_ref):
    z = jnp.dot(x_ref[...], w_ref[...], preferred_element_type=jnp.float32)
    x0 = jnp.maximum(z + b_ref[...], 0.0)
    deg = degs_ref[0, :, 0:1] + degs_ref[1, :, 0:1] + 1.0
    dinv = lax.rsqrt(deg)
    x0_ref[...] = x0
    dinv_ref[...] = dinv
    g = x0 * dinv
    g_ref[0] = g[:, :HH]
    g_ref[1] = g[:, HH:]


def _tc_prologue(x, lin_in_W, lin_in_b, degs):
    return pl.pallas_call(
        _tc_prologue_body,
        grid=(GRID,),
        in_specs=[
            pl.BlockSpec((BLK, GENE), lambda i: (i, 0)),
            pl.BlockSpec((GENE, H), lambda i: (0, 0)),
            pl.BlockSpec((1, H), lambda i: (0, 0)),
            pl.BlockSpec((NCORE, BLK, LANE), lambda i: (0, i, 0)),
        ],
        out_specs=[
            pl.BlockSpec((BLK, H), lambda i: (i, 0)),
            pl.BlockSpec((BLK, 1), lambda i: (i, 0)),
            pl.BlockSpec((NCORE, BLK, HH), lambda i: (0, i, 0)),
        ],
        out_shape=[
            jax.ShapeDtypeStruct((N, H), jnp.float32),
            jax.ShapeDtypeStruct((N, 1), jnp.float32),
            jax.ShapeDtypeStruct((NCORE, N, HH), jnp.float32),
        ],
    )(x, lin_in_W, lin_in_b, degs)


def _tc_layer_body(agg_ref, g_ref, x0_ref, dinv_ref, w_ref, s_ref, t_ref,
                   gn_ref, *, beta):
    a = jnp.concatenate(
        [agg_ref[0] + g_ref[0], agg_ref[1] + g_ref[1]], axis=1)
    dinv = dinv_ref[...]
    z = (1.0 - ALPHA) * (a * dinv) + ALPHA * x0_ref[...]
    zw = jnp.dot(z, w_ref[...], preferred_element_type=jnp.float32)
    out = jnp.maximum((1.0 - beta) * z + beta * zw, 0.0)
    h = out * (s_ref[...] * _INV_SQRT) + t_ref[...]
    g = h * dinv
    gn_ref[0] = g[:, :HH]
    gn_ref[1] = g[:, HH:]


def _tc_layer(agg, g, x0, dinv, W, s, t, beta):
    return pl.pallas_call(
        functools.partial(_tc_layer_body, beta=beta),
        grid=(GRID,),
        in_specs=[
            pl.BlockSpec((NCORE, BLK, HH), lambda i: (0, i, 0)),
            pl.BlockSpec((NCORE, BLK, HH), lambda i: (0, i, 0)),
            pl.BlockSpec((BLK, H), lambda i: (i, 0)),
            pl.BlockSpec((BLK, 1), lambda i: (i, 0)),
            pl.BlockSpec((H, H), lambda i: (0, 0)),
            pl.BlockSpec((1, H), lambda i: (0, 0)),
            pl.BlockSpec((1, H), lambda i: (0, 0)),
        ],
        out_specs=pl.BlockSpec((NCORE, BLK, HH), lambda i: (0, i, 0)),
        out_shape=jax.ShapeDtypeStruct((NCORE, N, HH), jnp.float32),
    )(agg, g, x0, dinv, W, s, t)


def _tc_final_body(agg_ref, g_ref, x0_ref, dinv_ref, w_ref, s_ref, t_ref,
                   wo_ref, bo_ref, o_ref, *, beta):
    a = jnp.concatenate(
        [agg_ref[0] + g_ref[0], agg_ref[1] + g_ref[1]], axis=1)
    dinv = dinv_ref[...]
    z = (1.0 - ALPHA) * (a * dinv) + ALPHA * x0_ref[...]
    zw = jnp.dot(z, w_ref[...], preferred_element_type=jnp.float32)
    out = jnp.maximum((1.0 - beta) * z + beta * zw, 0.0)
    h = out * (s_ref[...] * _INV_SQRT) + t_ref[...]
    o_ref[...] = jnp.dot(h, wo_ref[...],
                         preferred_element_type=jnp.float32) + bo_ref[...]


def _tc_final(agg, g, x0, dinv, W, s, t, Wo, bo, beta):
    return pl.pallas_call(
        functools.partial(_tc_final_body, beta=beta),
        grid=(GRID,),
        in_specs=[
            pl.BlockSpec((NCORE, BLK, HH), lambda i: (0, i, 0)),
            pl.BlockSpec((NCORE, BLK, HH), lambda i: (0, i, 0)),
            pl.BlockSpec((BLK, H), lambda i: (i, 0)),
            pl.BlockSpec((BLK, 1), lambda i: (i, 0)),
            pl.BlockSpec((H, H), lambda i: (0, 0)),
            pl.BlockSpec((1, H), lambda i: (0, 0)),
            pl.BlockSpec((1, H), lambda i: (0, 0)),
            pl.BlockSpec((H, NCLS), lambda i: (0, 0)),
            pl.BlockSpec((1, NCLS), lambda i: (0, 0)),
        ],
        out_specs=pl.BlockSpec((BLK, NCLS), lambda i: (i, 0)),
        out_shape=jax.ShapeDtypeStruct((N, NCLS), jnp.float32),
    )(agg, g, x0, dinv, W, s, t, Wo, bo)


def kernel(x, edge_index, lin_in_W, lin_in_b, W1, bn_gamma, bn_beta,
           lin_out_W, lin_out_b):
    src = edge_index[0].astype(jnp.int32)
    dst = edge_index[1].astype(jnp.int32)
    pad = E_PAD - E
    src_p = jnp.concatenate([src, jnp.zeros((pad,), jnp.int32)])
    dst_p = jnp.concatenate([dst, jnp.full((pad,), TRASH, jnp.int32)])
    src2 = jnp.concatenate([src_p, src_p + N]).reshape(2 * E_PAD // CHUNK, CHUNK)
    dst2 = dst_p.reshape(E_PAD // CHUNK, CHUNK)

    degs = _sc_deg(dst2)
    x0, dinv, g = _tc_prologue(x, lin_in_W, lin_in_b.reshape(1, H), degs)

    for i in range(L):
        beta = float(np.log(THETA / (i + 1) + 1.0))
        agg = _sc_agg(g.reshape(NCORE * N, HH), src2, dst2)
        s = bn_gamma[i].reshape(1, H)
        t = bn_beta[i].reshape(1, H)
        if i < L - 1:
            g = _tc_layer(agg, g, x0, dinv, W1[i], s, t, beta)
        else:
            out = _tc_final(agg, g, x0, dinv, W1[i], s, t,
                            lin_out_W, lin_out_b.reshape(1, NCLS), beta)
    return out


# speedup vs baseline: 5.7891x; 1.0700x over previous
import functools

import jax
import jax.numpy as jnp
import numpy as np
from jax import lax
from jax.experimental import pallas as pl
from jax.experimental.pallas import tpu as pltpu
from jax.experimental.pallas import tpu_sc as plsc

N = 10000
E = 160000
GENE = 256
H = 256
HH = 128
L = 8
NCLS = 16
ALPHA = 0.1
THETA = 0.5
EPS = 1e-5

NCORE = 2
NSUB = 16
LANE = 16

CHUNK = 128
E_PAD = 163840
EPC = E_PAD // NSUB
NCHUNK = EPC // CHUNK
N_ACC = 10240
RSUB = N_ACC // NSUB
TRASH = N_ACC - 1

DEG_NCHUNK = (E_PAD // 2) // NSUB // CHUNK

_mesh = plsc.VectorSubcoreMesh(
    core_axis_name="c", subcore_axis_name="s", num_cores=NCORE, num_subcores=NSUB)


def _sc_deg_body(dst2_hbm, out_hbm, dst_v, ones_v, zbuf, acc_sh):
    c = lax.axis_index("c")
    s = lax.axis_index("s")
    one16 = jnp.ones((LANE,), jnp.float32)
    zero16 = jnp.zeros((LANE,), jnp.float32)

    def initrow(i, _):
        ones_v[i, :] = one16
        return 0
    lax.fori_loop(0, CHUNK, initrow, 0)

    def zrow(i, _):
        zbuf[i, :] = zero16
        return 0
    lax.fori_loop(0, 64, zrow, 0)

    def zcp(k, _):
        pltpu.sync_copy(zbuf, acc_sh.at[pl.ds(s * RSUB + k * 64, 64)])
        return 0
    lax.fori_loop(0, RSUB // 64, zcp, 0)
    plsc.subcore_barrier()

    pltpu.sync_copy(
        dst2_hbm.at[pl.ds((c * NSUB + s) * DEG_NCHUNK, DEG_NCHUNK)], dst_v)

    def body(j, _):
        pltpu.sync_copy(ones_v, acc_sh.at[dst_v.at[j]], add=True)
        return 0
    lax.fori_loop(0, DEG_NCHUNK, body, 0)

    plsc.subcore_barrier()
    pltpu.sync_copy(acc_sh.at[pl.ds(s * RSUB, RSUB)],
                    out_hbm.at[c, pl.ds(s * RSUB, RSUB)])


_sc_deg = functools.partial(
    pl.kernel,
    out_type=jax.ShapeDtypeStruct((NCORE, N_ACC, LANE), jnp.float32),
    mesh=_mesh,
    scratch_types=[
        pltpu.VMEM((DEG_NCHUNK, CHUNK), jnp.int32),
        pltpu.VMEM((CHUNK, LANE), jnp.float32),
        pltpu.VMEM((64, LANE), jnp.float32),
        pltpu.VMEM_SHARED((N_ACC, LANE), jnp.float32),
    ],
)(_sc_deg_body)


def _sc_agg_body(g_hbm, src2_hbm, dst2_hbm, out_hbm,
                 src_v, dst_v, rows0, rows1, sem0, sem1, acc_sh):
    c = lax.axis_index("c")
    s = lax.axis_index("s")
    zero16 = jnp.zeros((LANE,), jnp.float32)

    def zrow(i, _):
        def zlane(j, _):
            rows0[i, pl.ds(j * LANE, LANE)] = zero16
            return 0
        return lax.fori_loop(0, HH // LANE, zlane, 0)
    lax.fori_loop(0, CHUNK, zrow, 0)

    def zcp(k, _):
        pltpu.sync_copy(rows0, acc_sh.at[pl.ds(s * RSUB + k * CHUNK, CHUNK)])
        return 0
    lax.fori_loop(0, RSUB // CHUNK, zcp, 0)
    plsc.subcore_barrier()

    half = NCHUNK // 2

    def phase(h, _):
        pltpu.sync_copy(
            src2_hbm.at[pl.ds((c * NSUB + s) * NCHUNK + h * half, half)], src_v)
        pltpu.sync_copy(
            dst2_hbm.at[pl.ds(s * NCHUNK + h * half, half)], dst_v)
        pltpu.async_copy(g_hbm.at[src_v.at[0]], rows0, sem0)

        def body(i, _):
            j0 = 2 * i
            j1 = 2 * i + 1
            pltpu.make_async_copy(g_hbm.at[src_v.at[j0]], rows0, sem0).wait()
            pltpu.async_copy(g_hbm.at[src_v.at[j1]], rows1, sem1)
            pltpu.sync_copy(rows0, acc_sh.at[dst_v.at[j0]], add=True)
            pltpu.make_async_copy(g_hbm.at[src_v.at[j1]], rows1, sem1).wait()

            @pl.when(i < half // 2 - 1)
            def _prefetch():
                pltpu.async_copy(g_hbm.at[src_v.at[j1 + 1]], rows0, sem0)
            pltpu.sync_copy(rows1, acc_sh.at[dst_v.at[j1]], add=True)
            return 0
        return lax.fori_loop(0, half // 2, body, 0)
    lax.fori_loop(0, 2, phase, 0)

    plsc.subcore_barrier()
    pltpu.sync_copy(acc_sh.at[pl.ds(s * RSUB, RSUB)],
                    out_hbm.at[c, pl.ds(s * RSUB, RSUB)])


_sc_agg = functools.partial(
    pl.kernel,
    out_type=jax.ShapeDtypeStruct((NCORE, N_ACC, HH), jnp.float32),
    mesh=_mesh,
    scratch_types=[
        pltpu.VMEM((NCHUNK // 2, CHUNK), jnp.int32),
        pltpu.VMEM((NCHUNK // 2, CHUNK), jnp.int32),
        pltpu.VMEM((CHUNK, HH), jnp.float32),
        pltpu.VMEM((CHUNK, HH), jnp.float32),
        pltpu.SemaphoreType.DMA,
        pltpu.SemaphoreType.DMA,
        pltpu.VMEM_SHARED((N_ACC, HH), jnp.float32),
    ],
)(_sc_agg_body)


BLK = 1000
GRID = N // BLK

_INV_SQRT = float(1.0 / np.sqrt(1.0 + EPS))


def _tc_prologue_body(x_ref, w_ref, b_ref, degs_ref, x0_ref, dinv_ref, g_ref):
    z = jnp.dot(x_ref[...], w_ref[...], preferred_element_type=jnp.float32)
    x0 = jnp.maximum(z + b_ref[...], 0.0)
    deg = degs_ref[0, :, 0:1] + degs_ref[1, :, 0:1] + 1.0
    dinv = lax.rsqrt(deg)
    x0_ref[...] = x0
    dinv_ref[...] = dinv
    g = x0 * dinv
    g_ref[0] = g[:, :HH]
    g_ref[1] = g[:, HH:]


def _tc_prologue(x, lin_in_W, lin_in_b, degs):
    return pl.pallas_call(
        _tc_prologue_body,
        grid=(GRID,),
        in_specs=[
            pl.BlockSpec((BLK, GENE), lambda i: (i, 0)),
            pl.BlockSpec((GENE, H), lambda i: (0, 0)),
            pl.BlockSpec((1, H), lambda i: (0, 0)),
            pl.BlockSpec((NCORE, BLK, LANE), lambda i: (0, i, 0)),
        ],
        out_specs=[
            pl.BlockSpec((BLK, H), lambda i: (i, 0)),
            pl.BlockSpec((BLK, 1), lambda i: (i, 0)),
            pl.BlockSpec((NCORE, BLK, HH), lambda i: (0, i, 0)),
        ],
        out_shape=[
            jax.ShapeDtypeStruct((N, H), jnp.float32),
            jax.ShapeDtypeStruct((N, 1), jnp.float32),
            jax.ShapeDtypeStruct((NCORE, N, HH), jnp.float32),
        ],
    )(x, lin_in_W, lin_in_b, degs)


def _tc_layer_body(agg_ref, g_ref, x0_ref, dinv_ref, w_ref, s_ref, t_ref,
                   gn_ref, *, beta):
    a = jnp.concatenate(
        [agg_ref[0] + g_ref[0], agg_ref[1] + g_ref[1]], axis=1)
    dinv = dinv_ref[...]
    z = (1.0 - ALPHA) * (a * dinv) + ALPHA * x0_ref[...]
    zw = jnp.dot(z, w_ref[...], preferred_element_type=jnp.float32)
    out = jnp.maximum((1.0 - beta) * z + beta * zw, 0.0)
    h = out * (s_ref[...] * _INV_SQRT) + t_ref[...]
    g = h * dinv
    gn_ref[0] = g[:, :HH]
    gn_ref[1] = g[:, HH:]


def _tc_layer(agg, g, x0, dinv, W, s, t, beta):
    return pl.pallas_call(
        functools.partial(_tc_layer_body, beta=beta),
        grid=(GRID,),
        in_specs=[
            pl.BlockSpec((NCORE, BLK, HH), lambda i: (0, i, 0)),
            pl.BlockSpec((NCORE, BLK, HH), lambda i: (0, i, 0)),
            pl.BlockSpec((BLK, H), lambda i: (i, 0)),
            pl.BlockSpec((BLK, 1), lambda i: (i, 0)),
            pl.BlockSpec((H, H), lambda i: (0, 0)),
            pl.BlockSpec((1, H), lambda i: (0, 0)),
            pl.BlockSpec((1, H), lambda i: (0, 0)),
        ],
        out_specs=pl.BlockSpec((NCORE, BLK, HH), lambda i: (0, i, 0)),
        out_shape=jax.ShapeDtypeStruct((NCORE, N, HH), jnp.float32),
    )(agg, g, x0, dinv, W, s, t)


def _tc_final_body(agg_ref, g_ref, x0_ref, dinv_ref, w_ref, s_ref, t_ref,
                   wo_ref, bo_ref, o_ref, *, beta):
    a = jnp.concatenate(
        [agg_ref[0] + g_ref[0], agg_ref[1] + g_ref[1]], axis=1)
    dinv = dinv_ref[...]
    z = (1.0 - ALPHA) * (a * dinv) + ALPHA * x0_ref[...]
    zw = jnp.dot(z, w_ref[...], preferred_element_type=jnp.float32)
    out = jnp.maximum((1.0 - beta) * z + beta * zw, 0.0)
    h = out * (s_ref[...] * _INV_SQRT) + t_ref[...]
    o_ref[...] = jnp.dot(h, wo_ref[...],
                         preferred_element_type=jnp.float32) + bo_ref[...]


def _tc_final(agg, g, x0, dinv, W, s, t, Wo, bo, beta):
    return pl.pallas_call(
        functools.partial(_tc_final_body, beta=beta),
        grid=(GRID,),
        in_specs=[
            pl.BlockSpec((NCORE, BLK, HH), lambda i: (0, i, 0)),
            pl.BlockSpec((NCORE, BLK, HH), lambda i: (0, i, 0)),
            pl.BlockSpec((BLK, H), lambda i: (i, 0)),
            pl.BlockSpec((BLK, 1), lambda i: (i, 0)),
            pl.BlockSpec((H, H), lambda i: (0, 0)),
            pl.BlockSpec((1, H), lambda i: (0, 0)),
            pl.BlockSpec((1, H), lambda i: (0, 0)),
            pl.BlockSpec((H, NCLS), lambda i: (0, 0)),
            pl.BlockSpec((1, NCLS), lambda i: (0, 0)),
        ],
        out_specs=pl.BlockSpec((BLK, NCLS), lambda i: (i, 0)),
        out_shape=jax.ShapeDtypeStruct((N, NCLS), jnp.float32),
    )(agg, g, x0, dinv, W, s, t, Wo, bo)


def kernel(x, edge_index, lin_in_W, lin_in_b, W1, bn_gamma, bn_beta,
           lin_out_W, lin_out_b):
    src = edge_index[0].astype(jnp.int32)
    dst = edge_index[1].astype(jnp.int32)
    pad = E_PAD - E
    src_p = jnp.concatenate([src, jnp.zeros((pad,), jnp.int32)])
    dst_p = jnp.concatenate([dst, jnp.full((pad,), TRASH, jnp.int32)])
    src2 = jnp.concatenate([src_p, src_p + N]).reshape(2 * E_PAD // CHUNK, CHUNK)
    dst2 = dst_p.reshape(E_PAD // CHUNK, CHUNK)

    degs = _sc_deg(dst2)
    x0, dinv, g = _tc_prologue(x, lin_in_W, lin_in_b.reshape(1, H), degs)

    for i in range(L):
        beta = float(np.log(THETA / (i + 1) + 1.0))
        agg = _sc_agg(g.reshape(NCORE * N, HH), src2, dst2)
        s = bn_gamma[i].reshape(1, H)
        t = bn_beta[i].reshape(1, H)
        if i < L - 1:
            g = _tc_layer(agg, g, x0, dinv, W1[i], s, t, beta)
        else:
            out = _tc_final(agg, g, x0, dinv, W1[i], s, t,
                            lin_out_W, lin_out_b.reshape(1, NCLS), beta)
    return out


# speedup vs baseline: 6.4389x; 1.1123x over previous
import functools

import jax
import jax.numpy as jnp
import numpy as np
from jax import lax
from jax.experimental import pallas as pl
from jax.experimental.pallas import tpu as pltpu
from jax.experimental.pallas import tpu_sc as plsc

N = 10000
E = 160000
GENE = 256
H = 256
HH = 128
L = 8
NCLS = 16
ALPHA = 0.1
THETA = 0.5
EPS = 1e-5

NCORE = 2
NSUB = 16
LANE = 16

CHUNK = 64
NBUF = 4
NPHASE = 4
E_PAD = 163840
EPC = E_PAD // NSUB
NCHUNK = EPC // CHUNK
N_ACC = 10240
RSUB = N_ACC // NSUB
TRASH = N_ACC - 1

DEG_NCHUNK = (E_PAD // 2) // NSUB // CHUNK

_mesh = plsc.VectorSubcoreMesh(
    core_axis_name="c", subcore_axis_name="s", num_cores=NCORE, num_subcores=NSUB)


def _sc_deg_body(dst2_hbm, out_hbm, dst_v, ones_v, zbuf, acc_sh):
    c = lax.axis_index("c")
    s = lax.axis_index("s")
    one16 = jnp.ones((LANE,), jnp.float32)
    zero16 = jnp.zeros((LANE,), jnp.float32)

    def initrow(i, _):
        ones_v[i, :] = one16
        return 0
    lax.fori_loop(0, CHUNK, initrow, 0)

    def zrow(i, _):
        zbuf[i, :] = zero16
        return 0
    lax.fori_loop(0, 64, zrow, 0)

    def zcp(k, _):
        pltpu.sync_copy(zbuf, acc_sh.at[pl.ds(s * RSUB + k * 64, 64)])
        return 0
    lax.fori_loop(0, RSUB // 64, zcp, 0)
    plsc.subcore_barrier()

    pltpu.sync_copy(
        dst2_hbm.at[pl.ds((c * NSUB + s) * DEG_NCHUNK, DEG_NCHUNK)], dst_v)

    def body(j, _):
        pltpu.sync_copy(ones_v, acc_sh.at[dst_v.at[j]], add=True)
        return 0
    lax.fori_loop(0, DEG_NCHUNK, body, 0)

    plsc.subcore_barrier()
    pltpu.sync_copy(acc_sh.at[pl.ds(s * RSUB, RSUB)],
                    out_hbm.at[c, pl.ds(s * RSUB, RSUB)])


_sc_deg = functools.partial(
    pl.kernel,
    out_type=jax.ShapeDtypeStruct((NCORE, N_ACC, LANE), jnp.float32),
    mesh=_mesh,
    scratch_types=[
        pltpu.VMEM((DEG_NCHUNK, CHUNK), jnp.int32),
        pltpu.VMEM((CHUNK, LANE), jnp.float32),
        pltpu.VMEM((64, LANE), jnp.float32),
        pltpu.VMEM_SHARED((N_ACC, LANE), jnp.float32),
    ],
)(_sc_deg_body)


def _sc_agg_body(g_hbm, src2_hbm, dst2_hbm, out_hbm,
                 src_v, dst_v, rows, sems, acc_sh):
    c = lax.axis_index("c")
    s = lax.axis_index("s")
    zero16 = jnp.zeros((LANE,), jnp.float32)

    def zrow(i, _):
        def zlane(j, _):
            rows[0][i, pl.ds(j * LANE, LANE)] = zero16
            return 0
        return lax.fori_loop(0, HH // LANE, zlane, 0)
    lax.fori_loop(0, CHUNK, zrow, 0)

    def zcp(k, _):
        pltpu.sync_copy(rows[0], acc_sh.at[pl.ds(s * RSUB + k * CHUNK, CHUNK)])
        return 0
    lax.fori_loop(0, RSUB // CHUNK, zcp, 0)
    plsc.subcore_barrier()

    pchunks = NCHUNK // NPHASE

    def phase(h, _):
        pltpu.sync_copy(
            src2_hbm.at[pl.ds((c * NSUB + s) * NCHUNK + h * pchunks, pchunks)],
            src_v)
        pltpu.sync_copy(
            dst2_hbm.at[pl.ds(s * NCHUNK + h * pchunks, pchunks)], dst_v)
        for k in range(NBUF - 1):
            pltpu.async_copy(g_hbm.at[src_v.at[k]], rows[k], sems[k])

        def body(i, _):
            for k in range(NBUF):
                j = NBUF * i + k
                kpre = (k + NBUF - 1) % NBUF
                pltpu.make_async_copy(
                    g_hbm.at[src_v.at[j]], rows[k], sems[k]).wait()

                @pl.when(j + NBUF - 1 < pchunks)
                def _prefetch():
                    pltpu.async_copy(g_hbm.at[src_v.at[j + NBUF - 1]],
                                     rows[kpre], sems[kpre])
                pltpu.sync_copy(rows[k], acc_sh.at[dst_v.at[j]], add=True)
            return 0
        return lax.fori_loop(0, pchunks // NBUF, body, 0)
    lax.fori_loop(0, NPHASE, phase, 0)

    plsc.subcore_barrier()
    pltpu.sync_copy(acc_sh.at[pl.ds(s * RSUB, RSUB)],
                    out_hbm.at[c, pl.ds(s * RSUB, RSUB)])


_sc_agg = functools.partial(
    pl.kernel,
    out_type=jax.ShapeDtypeStruct((NCORE, N_ACC, HH), jnp.float32),
    mesh=_mesh,
    scratch_types=[
        pltpu.VMEM((NCHUNK // NPHASE, CHUNK), jnp.int32),
        pltpu.VMEM((NCHUNK // NPHASE, CHUNK), jnp.int32),
        [pltpu.VMEM((CHUNK, HH), jnp.float32)] * NBUF,
        [pltpu.SemaphoreType.DMA] * NBUF,
        pltpu.VMEM_SHARED((N_ACC, HH), jnp.float32),
    ],
)(_sc_agg_body)


BLK = 1000
GRID = N // BLK

_INV_SQRT = float(1.0 / np.sqrt(1.0 + EPS))


def _tc_prologue_body(x_ref, w_ref, b_ref, degs_ref, x0_ref, dinv_ref, g_ref):
    z = jnp.dot(x_ref[...], w_ref[...], preferred_element_type=jnp.float32)
    x0 = jnp.maximum(z + b_ref[...], 0.0)
    deg = degs_ref[0, :, 0:1] + degs_ref[1, :, 0:1] + 1.0
    dinv = lax.rsqrt(deg)
    x0_ref[...] = x0
    dinv_ref[...] = dinv
    g = x0 * dinv
    g_ref[0] = g[:, :HH]
    g_ref[1] = g[:, HH:]


def _tc_prologue(x, lin_in_W, lin_in_b, degs):
    return pl.pallas_call(
        _tc_prologue_body,
        grid=(GRID,),
        in_specs=[
            pl.BlockSpec((BLK, GENE), lambda i: (i, 0)),
            pl.BlockSpec((GENE, H), lambda i: (0, 0)),
            pl.BlockSpec((1, H), lambda i: (0, 0)),
            pl.BlockSpec((NCORE, BLK, LANE), lambda i: (0, i, 0)),
        ],
        out_specs=[
            pl.BlockSpec((BLK, H), lambda i: (i, 0)),
            pl.BlockSpec((BLK, 1), lambda i: (i, 0)),
            pl.BlockSpec((NCORE, BLK, HH), lambda i: (0, i, 0)),
        ],
        out_shape=[
            jax.ShapeDtypeStruct((N, H), jnp.float32),
            jax.ShapeDtypeStruct((N, 1), jnp.float32),
            jax.ShapeDtypeStruct((NCORE, N, HH), jnp.float32),
        ],
    )(x, lin_in_W, lin_in_b, degs)


def _tc_layer_body(agg_ref, g_ref, x0_ref, dinv_ref, w_ref, s_ref, t_ref,
                   gn_ref, *, beta):
    a = jnp.concatenate(
        [agg_ref[0] + g_ref[0], agg_ref[1] + g_ref[1]], axis=1)
    dinv = dinv_ref[...]
    z = (1.0 - ALPHA) * (a * dinv) + ALPHA * x0_ref[...]
    zw = jnp.dot(z, w_ref[...], preferred_element_type=jnp.float32)
    out = jnp.maximum((1.0 - beta) * z + beta * zw, 0.0)
    h = out * (s_ref[...] * _INV_SQRT) + t_ref[...]
    g = h * dinv
    gn_ref[0] = g[:, :HH]
    gn_ref[1] = g[:, HH:]


def _tc_layer(agg, g, x0, dinv, W, s, t, beta):
    return pl.pallas_call(
        functools.partial(_tc_layer_body, beta=beta),
        grid=(GRID,),
        in_specs=[
            pl.BlockSpec((NCORE, BLK, HH), lambda i: (0, i, 0)),
            pl.BlockSpec((NCORE, BLK, HH), lambda i: (0, i, 0)),
            pl.BlockSpec((BLK, H), lambda i: (i, 0)),
            pl.BlockSpec((BLK, 1), lambda i: (i, 0)),
            pl.BlockSpec((H, H), lambda i: (0, 0)),
            pl.BlockSpec((1, H), lambda i: (0, 0)),
            pl.BlockSpec((1, H), lambda i: (0, 0)),
        ],
        out_specs=pl.BlockSpec((NCORE, BLK, HH), lambda i: (0, i, 0)),
        out_shape=jax.ShapeDtypeStruct((NCORE, N, HH), jnp.float32),
    )(agg, g, x0, dinv, W, s, t)


def _tc_final_body(agg_ref, g_ref, x0_ref, dinv_ref, w_ref, s_ref, t_ref,
                   wo_ref, bo_ref, o_ref, *, beta):
    a = jnp.concatenate(
        [agg_ref[0] + g_ref[0], agg_ref[1] + g_ref[1]], axis=1)
    dinv = dinv_ref[...]
    z = (1.0 - ALPHA) * (a * dinv) + ALPHA * x0_ref[...]
    zw = jnp.dot(z, w_ref[...], preferred_element_type=jnp.float32)
    out = jnp.maximum((1.0 - beta) * z + beta * zw, 0.0)
    h = out * (s_ref[...] * _INV_SQRT) + t_ref[...]
    o_ref[...] = jnp.dot(h, wo_ref[...],
                         preferred_element_type=jnp.float32) + bo_ref[...]


def _tc_final(agg, g, x0, dinv, W, s, t, Wo, bo, beta):
    return pl.pallas_call(
        functools.partial(_tc_final_body, beta=beta),
        grid=(GRID,),
        in_specs=[
            pl.BlockSpec((NCORE, BLK, HH), lambda i: (0, i, 0)),
            pl.BlockSpec((NCORE, BLK, HH), lambda i: (0, i, 0)),
            pl.BlockSpec((BLK, H), lambda i: (i, 0)),
            pl.BlockSpec((BLK, 1), lambda i: (i, 0)),
            pl.BlockSpec((H, H), lambda i: (0, 0)),
            pl.BlockSpec((1, H), lambda i: (0, 0)),
            pl.BlockSpec((1, H), lambda i: (0, 0)),
            pl.BlockSpec((H, NCLS), lambda i: (0, 0)),
            pl.BlockSpec((1, NCLS), lambda i: (0, 0)),
        ],
        out_specs=pl.BlockSpec((BLK, NCLS), lambda i: (i, 0)),
        out_shape=jax.ShapeDtypeStruct((N, NCLS), jnp.float32),
    )(agg, g, x0, dinv, W, s, t, Wo, bo)


def kernel(x, edge_index, lin_in_W, lin_in_b, W1, bn_gamma, bn_beta,
           lin_out_W, lin_out_b):
    src = edge_index[0].astype(jnp.int32)
    dst = edge_index[1].astype(jnp.int32)
    pad = E_PAD - E
    src_p = jnp.concatenate([src, jnp.zeros((pad,), jnp.int32)])
    dst_p = jnp.concatenate([dst, jnp.full((pad,), TRASH, jnp.int32)])
    src2 = jnp.concatenate([src_p, src_p + N]).reshape(2 * E_PAD // CHUNK, CHUNK)
    dst2 = dst_p.reshape(E_PAD // CHUNK, CHUNK)

    degs = _sc_deg(dst2)
    x0, dinv, g = _tc_prologue(x, lin_in_W, lin_in_b.reshape(1, H), degs)

    for i in range(L):
        beta = float(np.log(THETA / (i + 1) + 1.0))
        agg = _sc_agg(g.reshape(NCORE * N, HH), src2, dst2)
        s = bn_gamma[i].reshape(1, H)
        t = bn_beta[i].reshape(1, H)
        if i < L - 1:
            g = _tc_layer(agg, g, x0, dinv, W1[i], s, t, beta)
        else:
            out = _tc_final(agg, g, x0, dinv, W1[i], s, t,
                            lin_out_W, lin_out_b.reshape(1, NCLS), beta)
    return out
